# Initial kernel scaffold; baseline (speedup 1.0000x reference)
import jax
import jax.numpy as jnp
from jax.experimental import pallas as pl


def kernel(edge_attr, edge_index, num_nodes, W0, b0, W1, b1, Wout, bout):
    raise NotImplementedError("write your pallas kernel here")


# speedup vs baseline: 3.0051x; 3.0051x over previous
import functools

import numpy as np
import jax
import jax.numpy as jnp
from jax import lax
from jax.experimental import pallas as pl
from jax.experimental.pallas import tpu as pltpu
from jax.experimental.pallas import tpu_sc as plsc

_B, _E, _N, _D = 16, 16384, 1024, 128
_ET = 4096
_LOGEPS = float(np.log(np.float32(1e-10)))

_NC, _NS = 2, 16
_NW = _NC * _NS
_CELLS = _B * _N * _N
_REG = _CELLS // _NW
_CB = 16384
_NINIT = _REG // _CB
_CH = 128
_EPT = _B * _E // _NW
_NCH = _EPT // _CH
_EROWS = _B * _E // _CH


def _mlp_body(x_ref, ei_ref, w0_ref, b0_ref, w1_ref, b1_ref, wo_ref, bo_ref,
              val_ref, idx_ref):
    b = pl.program_id(0)
    x = x_ref[0]
    dn = (((1,), (1,)), ((), ()))
    h = lax.dot_general(x, w0_ref[...], dn, preferred_element_type=jnp.float32)
    h = jax.nn.silu(h + b0_ref[0])
    h = lax.dot_general(h, w1_ref[...], dn, preferred_element_type=jnp.float32)
    h = jax.nn.silu(h + b1_ref[0])
    z = lax.dot_general(wo_ref[...], h, dn,
                        preferred_element_type=jnp.float32) + bo_ref[0, 0]
    e = jax.nn.sigmoid(z)
    val_ref[...] = jnp.log(e + 1e-10)[:, None, :]
    src = ei_ref[0, 0:1]
    dst = ei_ref[0, 1:2]
    idx_ref[...] = (b * (_N * _N) + src * _N + dst)[:, None, :]


def _run_mlp(edge_attr, edge_index, W0, b0, W1, b1, Wout, bout):
    grid = (_B, _E // _ET)
    vals, idx = pl.pallas_call(
        _mlp_body,
        grid=grid,
        in_specs=[
            pl.BlockSpec((1, _ET, _D), lambda b, j: (b, j, 0)),
            pl.BlockSpec((1, 2, _ET), lambda b, j: (b, 0, j)),
            pl.BlockSpec((_D, _D), lambda b, j: (0, 0)),
            pl.BlockSpec((1, _D), lambda b, j: (0, 0)),
            pl.BlockSpec((_D, _D), lambda b, j: (0, 0)),
            pl.BlockSpec((1, _D), lambda b, j: (0, 0)),
            pl.BlockSpec((1, _D), lambda b, j: (0, 0)),
            pl.BlockSpec((1, 1), lambda b, j: (0, 0)),
        ],
        out_specs=[
            pl.BlockSpec((1, 1, _ET), lambda b, j: (b * (_E // _ET) + j, 0, 0)),
            pl.BlockSpec((1, 1, _ET), lambda b, j: (b * (_E // _ET) + j, 0, 0)),
        ],
        out_shape=[
            jax.ShapeDtypeStruct((_B * _E // _ET, 1, _ET), jnp.float32),
            jax.ShapeDtypeStruct((_B * _E // _ET, 1, _ET), jnp.int32),
        ],
    )(edge_attr, edge_index, W0, b0.reshape(1, _D), W1, b1.reshape(1, _D),
      Wout.reshape(1, _D), bout.reshape(1, 1))
    return vals, idx


_sc_mesh = plsc.VectorSubcoreMesh(core_axis_name="c", subcore_axis_name="s")


@functools.partial(
    pl.kernel,
    out_type=jax.ShapeDtypeStruct((_CELLS,), jnp.float32),
    mesh=_sc_mesh,
    scratch_types=[
        pltpu.VMEM((_CB,), jnp.float32),
        pltpu.VMEM((_NCH, _CH), jnp.int32),
        pltpu.VMEM((_NCH, _CH), jnp.float32),
        pltpu.SemaphoreType.DMA,
        pltpu.SemaphoreType.DMA,
    ],
)
def _sc_scatter(idx_hbm, val_hbm, out_hbm, cb, idx_v, val_v, sem_i, sem_s):
    c = lax.axis_index("c")
    s = lax.axis_index("s")
    w = c * _NS + s

    cvec = jnp.full((16,), _LOGEPS, jnp.float32)

    def fill(i, carry):
        cb[pl.ds(i * 64, 16)] = cvec
        cb[pl.ds(i * 64 + 16, 16)] = cvec
        cb[pl.ds(i * 64 + 32, 16)] = cvec
        cb[pl.ds(i * 64 + 48, 16)] = cvec
        return carry

    lax.fori_loop(0, _CB // 64, fill, 0)

    rb = c * (_EROWS // _NC) + s * _NCH
    pltpu.make_async_copy(idx_hbm.at[pl.ds(rb, _NCH)], idx_v, sem_s).start()
    pltpu.make_async_copy(val_hbm.at[pl.ds(rb, _NCH)], val_v, sem_s).start()

    base = w * _REG

    def fire_init(i, carry):
        pltpu.make_async_copy(
            cb, out_hbm.at[pl.ds(base + i * _CB, _CB)], sem_i).start()
        return carry

    lax.fori_loop(0, _NINIT, fire_init, 0)

    pltpu.make_async_copy(idx_hbm.at[pl.ds(rb, _NCH)], idx_v, sem_s).wait()
    pltpu.make_async_copy(val_hbm.at[pl.ds(rb, _NCH)], val_v, sem_s).wait()

    def drain_init(i, carry):
        pltpu.make_async_copy(
            cb, out_hbm.at[pl.ds(base + i * _CB, _CB)], sem_i).wait()
        return carry

    lax.fori_loop(0, _NINIT, drain_init, 0)

    plsc.subcore_barrier()

    def fire_scat(j, carry):
        pltpu.make_async_copy(
            val_v.at[j], out_hbm.at[idx_v.at[j]], sem_s).start()
        return carry

    lax.fori_loop(0, _NCH, fire_scat, 0)

    def drain_scat(j, carry):
        pltpu.make_async_copy(
            val_v.at[j], out_hbm.at[idx_v.at[j]], sem_s).wait()
        return carry

    lax.fori_loop(0, _NCH, drain_scat, 0)


def kernel(edge_attr, edge_index, num_nodes, W0, b0, W1, b1, Wout, bout):
    del num_nodes
    ei = edge_index.astype(jnp.int32)
    vals, idx = _run_mlp(edge_attr, ei, W0, b0, W1, b1, Wout, bout)
    idx2 = idx.reshape(_EROWS, _CH)
    vals2 = vals.reshape(_EROWS, _CH)
    flat = _sc_scatter(idx2, vals2)
    return flat.reshape(_B, _N, _N)
